# Initial kernel scaffold; baseline (speedup 1.0000x reference)
import jax
import jax.numpy as jnp
from jax.experimental import pallas as pl


def kernel(emb_emitter, emb_receiver):
    raise NotImplementedError("write your pallas kernel here")


# speedup vs baseline: 3.9171x; 3.9171x over previous
import jax
import jax.numpy as jnp
from jax.experimental import pallas as pl
from jax.experimental.pallas import tpu as pltpu

_N = 10000
_D = 128
_ALPHA = 3.0
_K = 16
_BR = 80
_CT = 1280
_NFULL = 7
_TILES = [(t * _CT, _CT) for t in range(_NFULL)] + [(_NFULL * _CT, _N - _NFULL * _CT)]


def _prep_kernel(e_ref, r_ref, m1_ref, m2_ref):
    m1_ref[...] = jnp.tanh(_ALPHA * e_ref[...]).astype(jnp.bfloat16)
    m2_ref[...] = jnp.tanh(_ALPHA * r_ref[...]).astype(jnp.bfloat16)


def _dot_nt(a, b):
    return jax.lax.dot_general(
        a, b, (((1,), (1,)), ((), ())),
        preferred_element_type=jnp.float32,
    )


def _main_kernel(m1b_ref, m2b_ref, m1_ref, m2_ref, out_ref, aw_ref):
    m1b = m1b_ref[...]
    m2b = m2b_ref[...]

    for off, w in _TILES:
        m1t = m1_ref[pl.ds(off, w), :]
        m2t = m2_ref[pl.ds(off, w), :]
        s = _dot_nt(m1b, m2t) - _dot_nt(m2b, m1t)
        aw_ref[:, pl.ds(off, w)] = jnp.maximum(jnp.tanh(_ALPHA * s), 0.0)

    def step(_, carry):
        m, f = carry
        best_v = jnp.full((_BR, 1), -1.0, dtype=jnp.float32)
        best_i = jnp.full((_BR, 1), _N, dtype=jnp.int32)
        for off, w in _TILES:
            a = aw_ref[:, pl.ds(off, w)]
            it = jax.lax.broadcasted_iota(jnp.int32, a.shape, 1) + off
            pred = (a < m) | ((a == m) & (it > f))
            v = jnp.where(pred, a, -1.0)
            mt = jnp.max(v, axis=1, keepdims=True)
            ft = jnp.min(jnp.where(v == mt, it, _N), axis=1, keepdims=True)
            upd = (mt > best_v) | ((mt == best_v) & (ft < best_i))
            best_i = jnp.where(upd, ft, best_i)
            best_v = jnp.where(upd, mt, best_v)
        return best_v, best_i

    m0 = jnp.full((_BR, 1), jnp.inf, dtype=jnp.float32)
    f0 = jnp.full((_BR, 1), -1, dtype=jnp.int32)
    m, f = jax.lax.fori_loop(0, _K, step, (m0, f0))

    for off, w in _TILES:
        a = aw_ref[:, pl.ds(off, w)]
        it = jax.lax.broadcasted_iota(jnp.int32, a.shape, 1) + off
        keep = (a > m) | ((a == m) & (it <= f))
        out_ref[:, pl.ds(off, w)] = jnp.where(keep, a, 0.0)


def kernel(emb_emitter, emb_receiver):
    m1, m2 = pl.pallas_call(
        _prep_kernel,
        out_shape=[jax.ShapeDtypeStruct((_N, _D), jnp.bfloat16)] * 2,
    )(emb_emitter, emb_receiver)

    out = pl.pallas_call(
        _main_kernel,
        grid=(_N // _BR,),
        in_specs=[
            pl.BlockSpec((_BR, _D), lambda i: (i, 0)),
            pl.BlockSpec((_BR, _D), lambda i: (i, 0)),
            pl.BlockSpec((_N, _D), lambda i: (0, 0)),
            pl.BlockSpec((_N, _D), lambda i: (0, 0)),
        ],
        out_specs=pl.BlockSpec((_BR, _N), lambda i: (i, 0)),
        out_shape=jax.ShapeDtypeStruct((_N, _N), jnp.float32),
        scratch_shapes=[pltpu.VMEM((_BR, _N), jnp.float32)],
        compiler_params=pltpu.CompilerParams(
            dimension_semantics=("parallel",),
        ),
    )(m1, m2, m1, m2)
    return out
